# baseline (speedup 1.0000x reference)
import functools

import jax
import jax.numpy as jnp
import numpy as np
from jax.experimental import pallas as pl

_TOPK = 64
_NEG = -1e9


def _qkv_kernel(x_ref, wq_ref, wk_ref, wv_ref, bq_ref, bk_ref, bv_ref,
                q_ref, k_ref, v_ref):
    x = x_ref[...]
    dn = (((1,), (1,)), ((), ()))
    for w_ref, b_ref, o_ref in ((wq_ref, bq_ref, q_ref),
                                (wk_ref, bk_ref, k_ref),
                                (wv_ref, bv_ref, v_ref)):
        acc = jax.lax.dot_general(x, w_ref[...], dn,
                                  preferred_element_type=jnp.float32)
        o_ref[...] = (acc + b_ref[...]).astype(jnp.bfloat16)


def _attn_kernel(q_ref, k_ref, v_ref, t_ref, o_ref, *, bq, topk, sqrt_d):
    i = pl.program_id(0)
    n = k_ref.shape[0]

    s = jax.lax.dot_general(
        q_ref[...], k_ref[...], (((1,), (1,)), ((), ())),
        preferred_element_type=jnp.float32)
    s = s / (t_ref[0, 0] * sqrt_d)

    rows = i * bq + jax.lax.broadcasted_iota(jnp.int32, (bq, n), 0)
    cols = jax.lax.broadcasted_iota(jnp.int32, (bq, n), 1)
    s = jnp.where(rows == cols, _NEG, s)

    s0 = jnp.where(s == 0.0, 0.0, s)
    bits = jax.lax.bitcast_convert_type(s0, jnp.int32)
    ks = bits ^ ((bits >> 31) & jnp.int32(0x7FFFFFFF))

    hi16 = (ks >> 16).astype(jnp.int16)
    half = jnp.int32(0x8000)

    def body_hi(it, p_u):
        bit = jax.lax.shift_left(jnp.int32(1), jnp.int32(15) - it)
        cand_u = p_u | bit
        cand_s = (cand_u - half).astype(jnp.int16)
        cnt = jnp.sum((hi16 >= cand_s).astype(jnp.int32), axis=1,
                      keepdims=True)
        return jnp.where(cnt >= topk, cand_u, p_u)

    p_u = jax.lax.fori_loop(0, 16, body_hi,
                            jnp.zeros((bq, 1), jnp.int32))
    p_s16 = (p_u - half).astype(jnp.int16)
    c_hi = jnp.sum((hi16 > p_s16).astype(jnp.int32), axis=1,
                   keepdims=True)
    lo_s = ((ks & jnp.int32(0xFFFF)) - half).astype(jnp.int16)
    lo_m = jnp.where(hi16 == p_s16, lo_s, jnp.int16(-32768))

    def body_lo(it, t_u):
        bit = jax.lax.shift_left(jnp.int32(1), jnp.int32(15) - it)
        cand_u = t_u | bit
        cand_s = (cand_u - half).astype(jnp.int16)
        cnt = c_hi + jnp.sum((lo_m >= cand_s).astype(jnp.int32), axis=1,
                             keepdims=True)
        return jnp.where(cnt >= topk, cand_u, t_u)

    t_u = jax.lax.fori_loop(0, 16, body_lo,
                            jnp.zeros((bq, 1), jnp.int32))
    t_key = (p_u - half) * 65536 + t_u
    keep = ks >= t_key

    s = jnp.where(keep, s, _NEG)
    m = jnp.max(s, axis=1, keepdims=True)
    p = jnp.exp(s - m)
    p = p / jnp.sum(p, axis=1, keepdims=True)

    o_ref[...] = jax.lax.dot_general(
        p.astype(jnp.bfloat16), v_ref[...],
        (((1,), (0,)), ((), ())), preferred_element_type=jnp.float32)


def kernel(example_features, Wq, bq, Wk, bk, Wv, bv, temperature):
    n, d = example_features.shape
    topk = _TOPK
    sqrt_d = float(np.sqrt(d).astype(np.float32))

    x16 = example_features.astype(jnp.bfloat16)
    w16 = [w.astype(jnp.bfloat16) for w in (Wq, Wk, Wv)]
    b2 = [b.reshape(1, d) for b in (bq, bk, bv)]

    bm = min(512, n)
    const = lambda i: (0, 0)
    qkv = pl.pallas_call(
        _qkv_kernel,
        grid=(n // bm,),
        in_specs=[pl.BlockSpec((bm, d), lambda i: (i, 0))]
        + [pl.BlockSpec((d, d), const)] * 3
        + [pl.BlockSpec((1, d), const)] * 3,
        out_specs=[pl.BlockSpec((bm, d), lambda i: (i, 0))] * 3,
        out_shape=[jax.ShapeDtypeStruct((n, d), jnp.bfloat16)] * 3,
    )(x16, *w16, *b2)

    bqk = min(256, n)
    temp = temperature.reshape(1, 1)
    out = pl.pallas_call(
        functools.partial(_attn_kernel, bq=bqk, topk=topk, sqrt_d=sqrt_d),
        grid=(n // bqk,),
        in_specs=[
            pl.BlockSpec((bqk, d), lambda i: (i, 0)),
            pl.BlockSpec((n, d), const),
            pl.BlockSpec((n, d), const),
            pl.BlockSpec((1, 1), const),
        ],
        out_specs=pl.BlockSpec((bqk, d), lambda i: (i, 0)),
        out_shape=jax.ShapeDtypeStruct((n, d), jnp.float32),
    )(qkv[0], qkv[1], qkv[2], temp)
    return out


# speedup vs baseline: 1.5155x; 1.5155x over previous
import functools

import jax
import jax.numpy as jnp
import numpy as np
from jax.experimental import pallas as pl

_TOPK = 64
_NEG = -1e9


def _qkv_kernel(x_ref, wq_ref, wk_ref, wv_ref, bq_ref, bk_ref, bv_ref,
                q_ref, k_ref, v_ref):
    x = x_ref[...].astype(jnp.bfloat16)
    dn = (((1,), (1,)), ((), ()))
    for w_ref, b_ref, o_ref in ((wq_ref, bq_ref, q_ref),
                                (wk_ref, bk_ref, k_ref),
                                (wv_ref, bv_ref, v_ref)):
        acc = jax.lax.dot_general(x, w_ref[...], dn,
                                  preferred_element_type=jnp.float32)
        o_ref[...] = (acc + b_ref[...]).astype(jnp.bfloat16)


def _attn_kernel(q_ref, k_ref, v_ref, t_ref, o_ref, *, bq, topk, sqrt_d):
    i = pl.program_id(0)
    n = k_ref.shape[0]

    s = jax.lax.dot_general(
        q_ref[...], k_ref[...], (((1,), (1,)), ((), ())),
        preferred_element_type=jnp.float32)
    s = s / (t_ref[0, 0] * sqrt_d)

    rows = i * bq + jax.lax.broadcasted_iota(jnp.int32, (bq, n), 0)
    cols = jax.lax.broadcasted_iota(jnp.int32, (bq, n), 1)
    s = jnp.where(rows == cols, _NEG, s)

    s0 = jnp.where(s == 0.0, 0.0, s)
    bits = jax.lax.bitcast_convert_type(s0, jnp.int32)
    ks = bits ^ ((bits >> 31) & jnp.int32(0x7FFFFFFF))

    sign = jnp.int32(-2147483648)

    def body(it, t_u):
        bit = jax.lax.shift_left(jnp.int32(1), jnp.int32(31) - it)
        cand_u = t_u | bit
        cand_s = cand_u ^ sign
        cnt = jnp.sum((ks >= cand_s).astype(jnp.int32), axis=1,
                      keepdims=True)
        return jnp.where(cnt >= topk, cand_u, t_u)

    t_u = jax.lax.fori_loop(0, 32, body, jnp.zeros((bq, 1), jnp.int32))
    keep = ks >= (t_u ^ sign)

    s = jnp.where(keep, s, _NEG)
    m = jnp.max(s, axis=1, keepdims=True)
    p = jnp.exp(s - m)
    p = p / jnp.sum(p, axis=1, keepdims=True)

    o_ref[...] = jax.lax.dot_general(
        p.astype(jnp.bfloat16), v_ref[...],
        (((1,), (0,)), ((), ())), preferred_element_type=jnp.float32)


def kernel(example_features, Wq, bq, Wk, bk, Wv, bv, temperature):
    n, d = example_features.shape
    topk = _TOPK
    sqrt_d = float(np.sqrt(d).astype(np.float32))

    w16 = [w.astype(jnp.bfloat16) for w in (Wq, Wk, Wv)]
    b2 = [b.reshape(1, d) for b in (bq, bk, bv)]

    bm = min(512, n)
    const = lambda i: (0, 0)
    qkv = pl.pallas_call(
        _qkv_kernel,
        grid=(n // bm,),
        in_specs=[pl.BlockSpec((bm, d), lambda i: (i, 0))]
        + [pl.BlockSpec((d, d), const)] * 3
        + [pl.BlockSpec((1, d), const)] * 3,
        out_specs=[pl.BlockSpec((bm, d), lambda i: (i, 0))] * 3,
        out_shape=[jax.ShapeDtypeStruct((n, d), jnp.bfloat16)] * 3,
    )(example_features, *w16, *b2)

    bqk = min(256, n)
    temp = temperature.reshape(1, 1)
    out = pl.pallas_call(
        functools.partial(_attn_kernel, bq=bqk, topk=topk, sqrt_d=sqrt_d),
        grid=(n // bqk,),
        in_specs=[
            pl.BlockSpec((bqk, d), lambda i: (i, 0)),
            pl.BlockSpec((n, d), const),
            pl.BlockSpec((n, d), const),
            pl.BlockSpec((1, 1), const),
        ],
        out_specs=pl.BlockSpec((bqk, d), lambda i: (i, 0)),
        out_shape=jax.ShapeDtypeStruct((n, d), jnp.float32),
    )(qkv[0], qkv[1], qkv[2], temp)
    return out


# speedup vs baseline: 1.5260x; 1.0069x over previous
import functools

import jax
import jax.numpy as jnp
import numpy as np
from jax.experimental import pallas as pl

_TOPK = 64
_NEG = -1e9


def _qkv_kernel(x_ref, wq_ref, wk_ref, wv_ref, bq_ref, bk_ref, bv_ref,
                q_ref, k_ref, v_ref):
    x = x_ref[...].astype(jnp.bfloat16)
    dn = (((1,), (1,)), ((), ()))
    for w_ref, b_ref, o_ref in ((wq_ref, bq_ref, q_ref),
                                (wk_ref, bk_ref, k_ref),
                                (wv_ref, bv_ref, v_ref)):
        acc = jax.lax.dot_general(x, w_ref[...], dn,
                                  preferred_element_type=jnp.float32)
        o_ref[...] = (acc + b_ref[...]).astype(jnp.bfloat16)


def _attn_kernel(q_ref, k_ref, v_ref, t_ref, o_ref, *, bq, topk, sqrt_d):
    i = pl.program_id(0)
    n = k_ref.shape[0]

    s = jax.lax.dot_general(
        q_ref[...], k_ref[...], (((1,), (1,)), ((), ())),
        preferred_element_type=jnp.float32)
    s = s / (t_ref[0, 0] * sqrt_d)

    rows = i * bq + jax.lax.broadcasted_iota(jnp.int32, (bq, n), 0)
    cols = jax.lax.broadcasted_iota(jnp.int32, (bq, n), 1)
    s = jnp.where(rows == cols, _NEG, s)

    s0 = jnp.where(s == 0.0, 0.0, s)
    bits = jax.lax.bitcast_convert_type(s0, jnp.int32)
    ks = bits ^ ((bits >> 31) & jnp.int32(0x7FFFFFFF))

    sign = jnp.int32(-2147483648)
    m_u = jnp.max(ks, axis=1, keepdims=True) ^ sign
    cm = jnp.max(ks.reshape(bq, topk, n // topk), axis=2)
    l_u = jnp.min(cm, axis=1, keepdims=True) ^ sign
    div = m_u ^ l_u
    for sh in (1, 2, 4, 8, 16):
        div = div | jax.lax.shift_right_logical(div, sh)
    t_init = m_u & ~div
    nbits = jnp.max(jax.lax.population_count(div))

    def body(it, t_u):
        bit = jax.lax.shift_left(jnp.int32(1), nbits - 1 - it)
        cand_u = t_u | bit
        cand_s = cand_u ^ sign
        cnt = jnp.sum((ks >= cand_s).astype(jnp.int32), axis=1,
                      keepdims=True)
        return jnp.where(cnt >= topk, cand_u, t_u)

    t_u = jax.lax.fori_loop(0, nbits, body, t_init)
    keep = ks >= (t_u ^ sign)

    s = jnp.where(keep, s, _NEG)
    m = jnp.max(s, axis=1, keepdims=True)
    p = jnp.exp(s - m)
    p = p / jnp.sum(p, axis=1, keepdims=True)

    o_ref[...] = jax.lax.dot_general(
        p.astype(jnp.bfloat16), v_ref[...],
        (((1,), (0,)), ((), ())), preferred_element_type=jnp.float32)


def kernel(example_features, Wq, bq, Wk, bk, Wv, bv, temperature):
    n, d = example_features.shape
    topk = _TOPK
    sqrt_d = float(np.sqrt(d).astype(np.float32))

    w16 = [w.astype(jnp.bfloat16) for w in (Wq, Wk, Wv)]
    b2 = [b.reshape(1, d) for b in (bq, bk, bv)]

    bm = min(512, n)
    const = lambda i: (0, 0)
    qkv = pl.pallas_call(
        _qkv_kernel,
        grid=(n // bm,),
        in_specs=[pl.BlockSpec((bm, d), lambda i: (i, 0))]
        + [pl.BlockSpec((d, d), const)] * 3
        + [pl.BlockSpec((1, d), const)] * 3,
        out_specs=[pl.BlockSpec((bm, d), lambda i: (i, 0))] * 3,
        out_shape=[jax.ShapeDtypeStruct((n, d), jnp.bfloat16)] * 3,
    )(example_features, *w16, *b2)

    bqk = min(256, n)
    temp = temperature.reshape(1, 1)
    out = pl.pallas_call(
        functools.partial(_attn_kernel, bq=bqk, topk=topk, sqrt_d=sqrt_d),
        grid=(n // bqk,),
        in_specs=[
            pl.BlockSpec((bqk, d), lambda i: (i, 0)),
            pl.BlockSpec((n, d), const),
            pl.BlockSpec((n, d), const),
            pl.BlockSpec((1, 1), const),
        ],
        out_specs=pl.BlockSpec((bqk, d), lambda i: (i, 0)),
        out_shape=jax.ShapeDtypeStruct((n, d), jnp.float32),
    )(qkv[0], qkv[1], qkv[2], temp)
    return out
